# baseline (speedup 1.0000x reference)
import jax
import jax.numpy as jnp
from jax.experimental import pallas as pl
from jax.experimental.pallas import tpu as pltpu

_T, _P, _O, _C, _F, _K = 16, 64, 4, 32, 64, 128
_TI = 2
_TO = 4


def _embed_body(x_ref, w_ref, b_ref, out_ref):
    j = pl.program_id(1)
    for t in range(_TI):
        for o in range(_O):
            xt = x_ref[t, :, o, :, :].transpose(1, 0, 2).reshape(_C * _P, _F)
            acc = jax.lax.dot_general(
                xt, w_ref[o], (((1,), (0,)), ((), ())),
                preferred_element_type=jnp.float32)
            acc = acc + b_ref[o][None, :]
            acc = jnp.maximum(acc, 0.1 * acc)
            out_ref[j * _TI + t, o * _C:(o + 1) * _C] = acc.reshape(_C, _P, _K)


def kernel(x, W, b):
    outs = pl.pallas_call(
        _embed_body,
        grid=(_T // _TO, _TO // _TI),
        in_specs=[
            pl.BlockSpec((_TI, _P, _O, _C, _F),
                         lambda i, j: (i * (_TO // _TI) + j, 0, 0, 0, 0)),
            pl.BlockSpec((_O, _F, _K), lambda i, j: (0, 0, 0)),
            pl.BlockSpec((_O, _K), lambda i, j: (0, 0)),
        ],
        out_specs=pl.BlockSpec((_TO, _O * _C, _P, _K),
                               lambda i, j: (i, 0, 0, 0)),
        out_shape=jax.ShapeDtypeStruct((_T, _O * _C, _P, _K), jnp.float32),
        compiler_params=pltpu.CompilerParams(
            dimension_semantics=("parallel", "arbitrary")),
    )(x, W, b)
    objCounts = jnp.full((_T, _P), _O * _C, dtype=jnp.int32)
    return outs, objCounts
